# baseline (speedup 1.0000x reference)
import functools
import math

import jax
import jax.numpy as jnp
from jax import lax
from jax.experimental import pallas as pl
from jax.experimental.pallas import tpu as pltpu
from jax.experimental.pallas import tpu_sc as plsc

N = 10000
NP = 10240
E = 320000
D_IN = 128
D_HID = 128
D_OUT = 40
D_OUTP = 128
BN_EPS = 1e-5
BN_INV = 1.0 / math.sqrt(1.0 + BN_EPS)

NC = 2
NS = 16
NW = NC * NS
CHUNK = 64
EPT_CH = 160
PH_CH = 40
NBUF = 4
EP = NW * EPT_CH * CHUNK
DEG_CH = 128
DEG_EPT = EP // (NW * DEG_CH)
ROWS_PT = NP // NS

BLK = 1024
GRID = NP // BLK

_MESH = plsc.VectorSubcoreMesh(core_axis_name="c", subcore_axis_name="s")


def _sc_degree_body(col_hbm, out_hbm, cnt_sh, cidx, ones_v, zeros_v):
    cid = lax.axis_index("c")
    sid = lax.axis_index("s")
    wid = cid * NS + sid
    for j in range(ROWS_PT // 16):
        zeros_v[pl.ds(j * 16, 16)] = jnp.zeros((16,), jnp.float32)
    for j in range(DEG_CH // 16):
        ones_v[pl.ds(j * 16, 16)] = jnp.ones((16,), jnp.float32)
    pltpu.sync_copy(zeros_v, cnt_sh.at[pl.ds(sid * ROWS_PT, ROWS_PT)])
    plsc.subcore_barrier()
    pltpu.sync_copy(col_hbm.at[wid], cidx)

    def body(j, carry):
        pltpu.sync_copy(ones_v, cnt_sh.at[cidx.at[j]], add=True)
        return carry

    lax.fori_loop(0, DEG_EPT, body, 0)
    plsc.subcore_barrier()
    pltpu.sync_copy(cnt_sh.at[pl.ds(sid * ROWS_PT, ROWS_PT)],
                    out_hbm.at[cid, pl.ds(sid * ROWS_PT, ROWS_PT)])


def _sc_degree(col3):
    return pl.kernel(
        _sc_degree_body,
        out_type=jax.ShapeDtypeStruct((NC, NP), jnp.float32),
        mesh=_MESH,
        scratch_types=[
            pltpu.VMEM_SHARED((NP,), jnp.float32),
            pltpu.VMEM((DEG_EPT, DEG_CH), jnp.int32),
            pltpu.VMEM((DEG_CH,), jnp.float32),
            pltpu.VMEM((ROWS_PT,), jnp.float32),
        ],
        name="sc_degree",
    )(col3)


def _sc_prop_body(hs_hbm, row_hbm, col_hbm, out_hbm,
                  acc_sh, ridx, cidx, b0, b1, b2, b3,
                  g0, g1, g2, g3, t0, t1, t2, t3, *, d):
    cid = lax.axis_index("c")
    sid = lax.axis_index("s")
    wid = cid * NS + sid
    dwords = d // 16
    bufs = (b0, b1, b2, b3)
    gsems = (g0, g1, g2, g3)
    ssems = (t0, t1, t2, t3)

    def zbody(i, carry):
        r = i // dwords
        c = (i % dwords) * 16
        b0[r, pl.ds(c, 16)] = jnp.zeros((16,), jnp.float32)
        return carry

    lax.fori_loop(0, CHUNK * dwords, zbody, 0)
    for k in range(ROWS_PT // CHUNK):
        pltpu.sync_copy(b0, acc_sh.at[pl.ds(sid * ROWS_PT + k * CHUNK, CHUNK)])
    plsc.subcore_barrier()

    for p in range(EPT_CH // PH_CH):
        pltpu.sync_copy(row_hbm.at[wid, pl.ds(p * PH_CH, PH_CH)], ridx)
        pltpu.sync_copy(col_hbm.at[wid, pl.ds(p * PH_CH, PH_CH)], cidx)
        for k in range(NBUF):
            pltpu.async_copy(hs_hbm.at[ridx.at[k]], bufs[k], gsems[k])

        def body(j2, carry):
            j = j2 * NBUF
            for k in range(NBUF):
                pltpu.make_async_copy(
                    hs_hbm.at[ridx.at[j + k]], bufs[k], gsems[k]).wait()
                pltpu.async_copy(bufs[k], acc_sh.at[cidx.at[j + k]],
                                 ssems[k], add=True)
            for k in range(NBUF):
                pltpu.make_async_copy(
                    bufs[k], acc_sh.at[cidx.at[j + k]], ssems[k]).wait()
                pltpu.async_copy(hs_hbm.at[ridx.at[j + k + NBUF]],
                                 bufs[k], gsems[k])
            return carry

        lax.fori_loop(0, PH_CH // NBUF - 1, body, 0)
        jt = PH_CH - NBUF
        for k in range(NBUF):
            pltpu.make_async_copy(
                hs_hbm.at[ridx.at[jt + k]], bufs[k], gsems[k]).wait()
            pltpu.async_copy(bufs[k], acc_sh.at[cidx.at[jt + k]],
                             ssems[k], add=True)
        for k in range(NBUF):
            pltpu.make_async_copy(
                bufs[k], acc_sh.at[cidx.at[jt + k]], ssems[k]).wait()
    plsc.subcore_barrier()
    pltpu.sync_copy(acc_sh.at[pl.ds(sid * ROWS_PT, ROWS_PT)],
                    out_hbm.at[cid, pl.ds(sid * ROWS_PT, ROWS_PT)])


def _sc_propagate(hs, row3, col3, d):
    return pl.kernel(
        functools.partial(_sc_prop_body, d=d),
        out_type=jax.ShapeDtypeStruct((NC, NP, d), jnp.float32),
        mesh=_MESH,
        scratch_types=(
            [pltpu.VMEM_SHARED((NP, d), jnp.float32),
             pltpu.VMEM((PH_CH, CHUNK), jnp.int32),
             pltpu.VMEM((PH_CH, CHUNK), jnp.int32)]
            + [pltpu.VMEM((CHUNK, d), jnp.float32)] * NBUF
            + [pltpu.SemaphoreType.DMA] * (2 * NBUF)
        ),
        name=f"sc_propagate_{d}",
    )(hs, row3, col3)


def _tc0_body(x_ref, w_ref, b_ref, cnt_ref, hs_ref):
    dis = lax.rsqrt(cnt_ref[0] + cnt_ref[1] + 1.0)
    h = lax.dot_general(x_ref[...], w_ref[...], (((1,), (1,)), ((), ())),
                        preferred_element_type=jnp.float32)
    hs_ref[...] = dis * (h + b_ref[...])


def _tc0(xp, W0, b0, cnt3):
    return pl.pallas_call(
        _tc0_body,
        grid=(GRID,),
        in_specs=[
            pl.BlockSpec((BLK, D_IN), lambda g: (g, 0)),
            pl.BlockSpec((D_HID, D_IN), lambda g: (0, 0)),
            pl.BlockSpec((1, D_HID), lambda g: (0, 0)),
            pl.BlockSpec((NC, BLK, 1), lambda g: (0, g, 0)),
        ],
        out_specs=pl.BlockSpec((BLK, D_HID), lambda g: (g, 0)),
        out_shape=jax.ShapeDtypeStruct((NP, D_HID), jnp.float32),
        name="tc_layer0",
    )(xp, W0, b0, cnt3)


def _tc_mid_body(s_ref, hs_ref, cnt_ref, g_ref, be_ref, w_ref, b_ref, o_ref):
    dis = lax.rsqrt(cnt_ref[0] + cnt_ref[1] + 1.0)
    z = dis * (s_ref[0] + s_ref[1] + hs_ref[...])
    z = z * (g_ref[...] * BN_INV) + be_ref[...]
    z = jnp.maximum(z, 0.0)
    h = lax.dot_general(z, w_ref[...], (((1,), (1,)), ((), ())),
                        preferred_element_type=jnp.float32)
    o_ref[...] = dis * (h + b_ref[...])


def _tc_mid(s, hs, cnt3, gamma, beta, W, b, d_out):
    return pl.pallas_call(
        _tc_mid_body,
        grid=(GRID,),
        in_specs=[
            pl.BlockSpec((NC, BLK, D_HID), lambda g: (0, g, 0)),
            pl.BlockSpec((BLK, D_HID), lambda g: (g, 0)),
            pl.BlockSpec((NC, BLK, 1), lambda g: (0, g, 0)),
            pl.BlockSpec((1, D_HID), lambda g: (0, 0)),
            pl.BlockSpec((1, D_HID), lambda g: (0, 0)),
            pl.BlockSpec((d_out, D_HID), lambda g: (0, 0)),
            pl.BlockSpec((1, d_out), lambda g: (0, 0)),
        ],
        out_specs=pl.BlockSpec((BLK, d_out), lambda g: (g, 0)),
        out_shape=jax.ShapeDtypeStruct((NP, d_out), jnp.float32),
        name=f"tc_mid_{d_out}",
    )(s, hs, cnt3, gamma, beta, W, b)


def _tc_final_body(s_ref, hs_ref, cnt_ref, o_ref):
    dis = lax.rsqrt(cnt_ref[0] + cnt_ref[1] + 1.0)
    z = dis * (s_ref[0] + s_ref[1] + hs_ref[...])
    colid = lax.broadcasted_iota(jnp.int32, (BLK, D_OUTP), 1)
    zm = jnp.where(colid < D_OUT, z, -jnp.inf)
    m = jnp.max(zm, axis=1, keepdims=True)
    lse = jnp.log(jnp.sum(jnp.exp(zm - m), axis=1, keepdims=True)) + m
    o_ref[...] = zm - lse


def _tc_final(s, hs, cnt3):
    return pl.pallas_call(
        _tc_final_body,
        grid=(GRID,),
        in_specs=[
            pl.BlockSpec((NC, BLK, D_OUTP), lambda g: (0, g, 0)),
            pl.BlockSpec((BLK, D_OUTP), lambda g: (g, 0)),
            pl.BlockSpec((NC, BLK, 1), lambda g: (0, g, 0)),
        ],
        out_specs=pl.BlockSpec((BLK, D_OUTP), lambda g: (g, 0)),
        out_shape=jax.ShapeDtypeStruct((NP, D_OUTP), jnp.float32),
        name="tc_final",
    )(s, hs, cnt3)


def kernel(x, edge_index, W0, b0, gamma0, beta0, W1, b1, gamma1, beta1, W2, b2):
    f32 = jnp.float32
    xp = jnp.pad(x.astype(f32), ((0, NP - N), (0, 0)))
    row = edge_index[0].astype(jnp.int32)
    col = edge_index[1].astype(jnp.int32)
    npad = EP - E
    pad_i = jnp.arange(npad, dtype=jnp.int32)
    pad_r = pad_i % N
    pad_c = N + pad_i % (NP - N)
    row3 = jnp.concatenate([row, pad_r]).reshape(NW, EPT_CH, CHUNK)
    colf = jnp.concatenate([col, pad_c])
    col3 = colf.reshape(NW, EPT_CH, CHUNK)
    col3d = colf.reshape(NW, DEG_EPT, DEG_CH)
    W2p = jnp.pad(W2, ((0, D_OUTP - D_OUT), (0, 0)))
    b2p = jnp.pad(b2, ((0, D_OUTP - D_OUT),))

    cnt = _sc_degree(col3d)
    cnt3 = cnt.reshape(NC, NP, 1)

    hs0 = _tc0(xp, W0, b0.reshape(1, -1), cnt3)
    s0 = _sc_propagate(hs0, row3, col3, D_HID)
    hs1 = _tc_mid(s0, hs0, cnt3, gamma0.reshape(1, -1), beta0.reshape(1, -1),
                  W1, b1.reshape(1, -1), D_HID)
    s1 = _sc_propagate(hs1, row3, col3, D_HID)
    hs2 = _tc_mid(s1, hs1, cnt3, gamma1.reshape(1, -1), beta1.reshape(1, -1),
                  W2p, b2p.reshape(1, -1), D_OUTP)
    s2 = _sc_propagate(hs2, row3, col3, D_OUTP)
    outp = _tc_final(s2, hs2, cnt3)
    return outp[:N, :D_OUT]


# speedup vs baseline: 1.1291x; 1.1291x over previous
import functools
import math

import jax
import jax.numpy as jnp
from jax import lax
from jax.experimental import pallas as pl
from jax.experimental.pallas import tpu as pltpu
from jax.experimental.pallas import tpu_sc as plsc

N = 10000
NP = 10240
E = 320000
D_IN = 128
D_HID = 128
D_OUT = 40
D_OUTP = 128
BN_EPS = 1e-5
BN_INV = 1.0 / math.sqrt(1.0 + BN_EPS)

NC = 2
NS = 16
NW = NC * NS
CHUNK = 64
EPT_CH = 160
PH_CH = 40
NBUF = 4
EP = NW * EPT_CH * CHUNK
DEG_CH = 128
DEG_EPT = EP // (NW * DEG_CH)
ROWS_PT = NP // NS

BLK = 1000
GRID = N // BLK

_MESH = plsc.VectorSubcoreMesh(core_axis_name="c", subcore_axis_name="s")


def _sc_degree_body(col_hbm, out_hbm, cnt_sh, cidx, ones_v, zeros_v):
    cid = lax.axis_index("c")
    sid = lax.axis_index("s")
    wid = cid * NS + sid
    for j in range(ROWS_PT // 16):
        zeros_v[pl.ds(j * 16, 16)] = jnp.zeros((16,), jnp.float32)
    for j in range(DEG_CH // 16):
        ones_v[pl.ds(j * 16, 16)] = jnp.ones((16,), jnp.float32)
    pltpu.sync_copy(zeros_v, cnt_sh.at[pl.ds(sid * ROWS_PT, ROWS_PT)])
    plsc.subcore_barrier()
    pltpu.sync_copy(col_hbm.at[wid], cidx)

    def body(j, carry):
        pltpu.sync_copy(ones_v, cnt_sh.at[cidx.at[j]], add=True)
        return carry

    lax.fori_loop(0, DEG_EPT, body, 0)
    plsc.subcore_barrier()
    pltpu.sync_copy(cnt_sh.at[pl.ds(sid * ROWS_PT, ROWS_PT)],
                    out_hbm.at[cid, pl.ds(sid * ROWS_PT, ROWS_PT)])


def _sc_degree(col3):
    return pl.kernel(
        _sc_degree_body,
        out_type=jax.ShapeDtypeStruct((NC, NP), jnp.float32),
        mesh=_MESH,
        scratch_types=[
            pltpu.VMEM_SHARED((NP,), jnp.float32),
            pltpu.VMEM((DEG_EPT, DEG_CH), jnp.int32),
            pltpu.VMEM((DEG_CH,), jnp.float32),
            pltpu.VMEM((ROWS_PT,), jnp.float32),
        ],
        name="sc_degree",
    )(col3)


def _sc_prop_body(hs_hbm, row_hbm, col_hbm, out_hbm,
                  acc_sh, ridx, cidx, b0, b1, b2, b3,
                  g0, g1, g2, g3, *, d):
    cid = lax.axis_index("c")
    sid = lax.axis_index("s")
    wid = cid * NS + sid
    dwords = d // 16
    bufs = (b0, b1, b2, b3)
    gsems = (g0, g1, g2, g3)

    def zbody(i, carry):
        r = i // dwords
        c = (i % dwords) * 16
        b0[r, pl.ds(c, 16)] = jnp.zeros((16,), jnp.float32)
        return carry

    lax.fori_loop(0, CHUNK * dwords, zbody, 0)
    for k in range(ROWS_PT // CHUNK):
        pltpu.sync_copy(b0, acc_sh.at[pl.ds(sid * ROWS_PT + k * CHUNK, CHUNK)])
    plsc.subcore_barrier()

    for p in range(EPT_CH // PH_CH):
        pltpu.sync_copy(row_hbm.at[wid, pl.ds(p * PH_CH, PH_CH)], ridx)
        pltpu.sync_copy(col_hbm.at[wid, pl.ds(p * PH_CH, PH_CH)], cidx)
        for k in range(NBUF):
            pltpu.async_copy(hs_hbm.at[ridx.at[k]], bufs[k], gsems[k])

        def body(j2, carry):
            j = j2 * NBUF
            for k in range(NBUF):
                pltpu.make_async_copy(
                    hs_hbm.at[ridx.at[j + k]], bufs[k], gsems[k]).wait()
                pltpu.sync_copy(bufs[k], acc_sh.at[cidx.at[j + k]], add=True)
                pltpu.async_copy(hs_hbm.at[ridx.at[j + k + NBUF]],
                                 bufs[k], gsems[k])
            return carry

        lax.fori_loop(0, PH_CH // NBUF - 1, body, 0)
        jt = PH_CH - NBUF
        for k in range(NBUF):
            pltpu.make_async_copy(
                hs_hbm.at[ridx.at[jt + k]], bufs[k], gsems[k]).wait()
            pltpu.sync_copy(bufs[k], acc_sh.at[cidx.at[jt + k]], add=True)
    plsc.subcore_barrier()
    pltpu.sync_copy(acc_sh.at[pl.ds(sid * ROWS_PT, ROWS_PT)],
                    out_hbm.at[cid, pl.ds(sid * ROWS_PT, ROWS_PT)])


def _sc_propagate(hs, row3, col3, d):
    return pl.kernel(
        functools.partial(_sc_prop_body, d=d),
        out_type=jax.ShapeDtypeStruct((NC, NP, d), jnp.float32),
        mesh=_MESH,
        scratch_types=(
            [pltpu.VMEM_SHARED((NP, d), jnp.float32),
             pltpu.VMEM((PH_CH, CHUNK), jnp.int32),
             pltpu.VMEM((PH_CH, CHUNK), jnp.int32)]
            + [pltpu.VMEM((CHUNK, d), jnp.float32)] * NBUF
            + [pltpu.SemaphoreType.DMA] * NBUF
        ),
        name=f"sc_propagate_{d}",
    )(hs, row3, col3)


def _tc0_body(x_ref, w_ref, b_ref, cnt_ref, hs_ref):
    dis = lax.rsqrt(cnt_ref[0] + cnt_ref[1] + 1.0)
    h = lax.dot_general(x_ref[...], w_ref[...], (((1,), (1,)), ((), ())),
                        preferred_element_type=jnp.float32)
    hs_ref[...] = dis * (h + b_ref[...])


def _tc0(x, W0, b0, cnt3):
    return pl.pallas_call(
        _tc0_body,
        grid=(GRID,),
        in_specs=[
            pl.BlockSpec((BLK, D_IN), lambda g: (g, 0)),
            pl.BlockSpec((D_HID, D_IN), lambda g: (0, 0)),
            pl.BlockSpec((1, D_HID), lambda g: (0, 0)),
            pl.BlockSpec((NC, BLK, 1), lambda g: (0, g, 0)),
        ],
        out_specs=pl.BlockSpec((BLK, D_HID), lambda g: (g, 0)),
        out_shape=jax.ShapeDtypeStruct((N, D_HID), jnp.float32),
        name="tc_layer0",
    )(x, W0, b0, cnt3)


def _tc_mid_body(s_ref, hs_ref, cnt_ref, g_ref, be_ref, w_ref, b_ref, o_ref):
    dis = lax.rsqrt(cnt_ref[0] + cnt_ref[1] + 1.0)
    z = dis * (s_ref[0] + s_ref[1] + hs_ref[...])
    z = z * (g_ref[...] * BN_INV) + be_ref[...]
    z = jnp.maximum(z, 0.0)
    h = lax.dot_general(z, w_ref[...], (((1,), (1,)), ((), ())),
                        preferred_element_type=jnp.float32)
    o_ref[...] = dis * (h + b_ref[...])


def _tc_mid(s, hs, cnt3, gamma, beta, W, b, d_out):
    return pl.pallas_call(
        _tc_mid_body,
        grid=(GRID,),
        in_specs=[
            pl.BlockSpec((NC, BLK, D_HID), lambda g: (0, g, 0)),
            pl.BlockSpec((BLK, D_HID), lambda g: (g, 0)),
            pl.BlockSpec((NC, BLK, 1), lambda g: (0, g, 0)),
            pl.BlockSpec((1, D_HID), lambda g: (0, 0)),
            pl.BlockSpec((1, D_HID), lambda g: (0, 0)),
            pl.BlockSpec((d_out, D_HID), lambda g: (0, 0)),
            pl.BlockSpec((1, d_out), lambda g: (0, 0)),
        ],
        out_specs=pl.BlockSpec((BLK, d_out), lambda g: (g, 0)),
        out_shape=jax.ShapeDtypeStruct((N, d_out), jnp.float32),
        name=f"tc_mid_{d_out}",
    )(s, hs, cnt3, gamma, beta, W, b)


def _tc_final_body(s_ref, hs_ref, cnt_ref, o_ref):
    dis = lax.rsqrt(cnt_ref[0] + cnt_ref[1] + 1.0)
    z = dis * (s_ref[0] + s_ref[1] + hs_ref[...])
    colid = lax.broadcasted_iota(jnp.int32, (BLK, D_OUTP), 1)
    zm = jnp.where(colid < D_OUT, z, -jnp.inf)
    m = jnp.max(zm, axis=1, keepdims=True)
    lse = jnp.log(jnp.sum(jnp.exp(zm - m), axis=1, keepdims=True)) + m
    o_ref[...] = (zm - lse)[:, :D_OUT]


def _tc_final(s, hs, cnt3):
    return pl.pallas_call(
        _tc_final_body,
        grid=(GRID,),
        in_specs=[
            pl.BlockSpec((NC, BLK, D_OUTP), lambda g: (0, g, 0)),
            pl.BlockSpec((BLK, D_OUTP), lambda g: (g, 0)),
            pl.BlockSpec((NC, BLK, 1), lambda g: (0, g, 0)),
        ],
        out_specs=pl.BlockSpec((BLK, D_OUT), lambda g: (g, 0)),
        out_shape=jax.ShapeDtypeStruct((N, D_OUT), jnp.float32),
        name="tc_final",
    )(s, hs, cnt3)


def kernel(x, edge_index, W0, b0, gamma0, beta0, W1, b1, gamma1, beta1, W2, b2):
    f32 = jnp.float32
    x = x.astype(f32)
    row = edge_index[0].astype(jnp.int32)
    col = edge_index[1].astype(jnp.int32)
    npad = EP - E
    pad_i = jnp.arange(npad, dtype=jnp.int32)
    pad_r = pad_i % N
    pad_c = N + pad_i % (NP - N)
    row3 = jnp.concatenate([row, pad_r]).reshape(NW, EPT_CH, CHUNK)
    colf = jnp.concatenate([col, pad_c])
    col3 = colf.reshape(NW, EPT_CH, CHUNK)
    col3d = colf.reshape(NW, DEG_EPT, DEG_CH)
    W2p = jnp.pad(W2, ((0, D_OUTP - D_OUT), (0, 0)))
    b2p = jnp.pad(b2, ((0, D_OUTP - D_OUT),))

    cnt = _sc_degree(col3d)
    cnt3 = cnt.reshape(NC, NP, 1)

    hs0 = _tc0(x, W0, b0.reshape(1, -1), cnt3)
    s0 = _sc_propagate(hs0, row3, col3, D_HID)
    hs1 = _tc_mid(s0, hs0, cnt3, gamma0.reshape(1, -1), beta0.reshape(1, -1),
                  W1, b1.reshape(1, -1), D_HID)
    s1 = _sc_propagate(hs1, row3, col3, D_HID)
    hs2 = _tc_mid(s1, hs1, cnt3, gamma1.reshape(1, -1), beta1.reshape(1, -1),
                  W2p, b2p.reshape(1, -1), D_OUTP)
    s2 = _sc_propagate(hs2, row3, col3, D_OUTP)
    return _tc_final(s2, hs2, cnt3)
